# Initial kernel scaffold; baseline (speedup 1.0000x reference)
import jax
import jax.numpy as jnp
from jax.experimental import pallas as pl


def kernel(x, edge_index, batch, w_l1, b_l1, w_r1, b_r1, att1, bias1, w_l2, b_l2, w_r2, b_r2, att2, bias2, w_l3, b_l3, w_r3, b_r3, att3, bias3, norm_w, norm_b, cls_w, cls_b):
    raise NotImplementedError("write your pallas kernel here")


# speedup vs baseline: 15.0039x; 15.0039x over previous
import functools

import jax
import jax.numpy as jnp
from jax import lax
from jax.experimental import pallas as pl
from jax.experimental.pallas import tpu as pltpu
from jax.experimental.pallas import tpu_sc as plsc

N = 10000
NP = 10240
NA = 10112
NAS = NA // 16
E = 170000
EP = 196608
EPW = EP // 32
CH = EPW // 128
NB = NP // 512
EB = 336
F32 = jnp.float32

_MESH = dict(core_axis_name="c", subcore_axis_name="s", num_cores=2,
             num_subcores=16)


@functools.cache
def _make_gather(W):
    mesh = plsc.VectorSubcoreMesh(**_MESH)

    @functools.partial(
        pl.kernel, mesh=mesh,
        out_type=(jax.ShapeDtypeStruct((EP, W), F32),
                  jax.ShapeDtypeStruct((EP, W), F32)),
        scratch_types=[
            pltpu.VMEM((CH, 128), jnp.int32),
            pltpu.VMEM((CH, 128), jnp.int32),
            pltpu.VMEM((128, W), F32),
            pltpu.VMEM((128, W), F32),
            pltpu.SemaphoreType.DMA,
            pltpu.SemaphoreType.DMA,
        ])
    def gather(xl_hbm, xr_hbm, src_hbm, dst_hbm, xj_out, xi_out,
               sidx, didx, jbuf, ibuf, sem1, sem2):
        wid = lax.axis_index("s") * 2 + lax.axis_index("c")
        row0 = wid * CH
        base = wid * EPW
        pltpu.sync_copy(src_hbm.at[pl.ds(row0, CH)], sidx)
        pltpu.sync_copy(dst_hbm.at[pl.ds(row0, CH)], didx)

        def body(t, carry):
            @pl.when(base + t * 128 < E)
            def _():
                cj = pltpu.async_copy(xl_hbm.at[sidx.at[t]], jbuf, sem1)
                ci = pltpu.async_copy(xr_hbm.at[didx.at[t]], ibuf, sem2)
                cj.wait()
                pltpu.sync_copy(jbuf, xj_out.at[pl.ds(base + t * 128, 128)])
                ci.wait()
                pltpu.sync_copy(ibuf, xi_out.at[pl.ds(base + t * 128, 128)])
            return carry

        lax.fori_loop(0, CH, body, 0)

    return gather


@functools.cache
def _make_scatter(nslab):
    mesh = plsc.VectorSubcoreMesh(**_MESH)

    @functools.partial(
        pl.kernel, mesh=mesh,
        out_type=jax.ShapeDtypeStruct((2, nslab, NA, 128), F32),
        scratch_types=[
            pltpu.VMEM((CH, 128), jnp.int32),
            pltpu.VMEM((128, 128), F32),
            pltpu.VMEM_SHARED((NA, 128), F32),
        ])
    def scatter(*args):
        m_hbms = args[:nslab]
        dst_hbm, zero_hbm, out_hbm, didx, mbuf, acc = args[nslab:]
        c = lax.axis_index("c")
        s = lax.axis_index("s")
        wid = s * 2 + c
        off = s * NAS
        row0 = wid * CH
        base = wid * EPW
        pltpu.sync_copy(dst_hbm.at[pl.ds(row0, CH)], didx)
        for k, m_hbm in enumerate(m_hbms):
            pltpu.sync_copy(zero_hbm.at[pl.ds(off, NAS)],
                            acc.at[pl.ds(off, NAS)])
            plsc.subcore_barrier()

            def body(t, carry, m_hbm=m_hbm):
                @pl.when(base + t * 128 < E)
                def _():
                    pltpu.sync_copy(m_hbm.at[pl.ds(base + t * 128, 128)],
                                    mbuf)
                    pltpu.sync_copy(mbuf, acc.at[didx.at[t]], add=True)
                return carry

            lax.fori_loop(0, CH, body, 0)
            plsc.subcore_barrier()
            pltpu.sync_copy(acc.at[pl.ds(off, NAS)],
                            out_hbm.at[c, k, pl.ds(off, NAS)])
            if k + 1 < nslab:
                plsc.subcore_barrier()

    return scatter


def _lin2_body(x_ref, wl_ref, bl_ref, wr_ref, br_ref, xl_ref, xr_ref):
    x = x_ref[...]
    xl_ref[...] = jnp.dot(x, wl_ref[...],
                          preferred_element_type=F32) + bl_ref[...]
    xr_ref[...] = jnp.dot(x, wr_ref[...],
                          preferred_element_type=F32) + br_ref[...]


def _lin2(x, wl, bl, wr, br):
    fi = x.shape[1]
    fo = wl.shape[1]
    return pl.pallas_call(
        _lin2_body, grid=(NB,),
        in_specs=[
            pl.BlockSpec((512, fi), lambda i: (i, 0)),
            pl.BlockSpec((fi, fo), lambda i: (0, 0)),
            pl.BlockSpec((1, fo), lambda i: (0, 0)),
            pl.BlockSpec((fi, fo), lambda i: (0, 0)),
            pl.BlockSpec((1, fo), lambda i: (0, 0)),
        ],
        out_specs=[pl.BlockSpec((512, fo), lambda i: (i, 0)),
                   pl.BlockSpec((512, fo), lambda i: (i, 0))],
        out_shape=[jax.ShapeDtypeStruct((NP, fo), F32),
                   jax.ShapeDtypeStruct((NP, fo), F32)],
    )(x, wl, bl.reshape(1, fo), wr, br.reshape(1, fo))


def _expack(es, w):
    col = lax.broadcasted_iota(jnp.int32, (512, w), 1)
    out = jnp.zeros((512, w), F32)
    for h, eh in enumerate(es):
        out = jnp.where(col == h, eh, out)
    return out


def _make_edge_body(oc, win):
    def body(xj_ref, xi_ref, att_ref, *out_refs):
        xj = xj_ref[...]
        z = xj + xi_ref[...]
        t = jnp.where(z >= 0, z, 0.2 * z) * att_ref[...]
        es = []
        for h in range(3):
            sl = slice(h * oc, (h + 1) * oc)
            es.append(jnp.exp(jnp.sum(t[:, sl], axis=1, keepdims=True)))
        if oc == 64:
            out_refs[0][...] = jnp.concatenate(
                [xj[:, 0:64] * es[0], xj[:, 64:128] * es[1]], axis=1)
            out_refs[1][...] = jnp.concatenate(
                [xj[:, 128:192] * es[2], _expack(es, 64)], axis=1)
        else:
            for h in range(3):
                out_refs[h][...] = xj[:, h * 128:(h + 1) * 128] * es[h]
            out_refs[3][...] = _expack(es, 128)
    return body


def _edge(xj, xi, att_flat, oc):
    win = xj.shape[1]
    nslab = 2 if oc == 64 else 4
    return pl.pallas_call(
        _make_edge_body(oc, win), grid=(EB,),
        in_specs=[
            pl.BlockSpec((512, win), lambda i: (i, 0)),
            pl.BlockSpec((512, win), lambda i: (i, 0)),
            pl.BlockSpec((1, win), lambda i: (0, 0)),
        ],
        out_specs=[pl.BlockSpec((512, 128), lambda i: (i, 0))] * nslab,
        out_shape=[jax.ShapeDtypeStruct((EP, 128), F32)] * nslab,
    )(xj, xi, att_flat)


def _norm_msgs(mp, bias):
    a = mp[0, 0] + mp[1, 0]
    b = mp[0, 1] + mp[1, 1]
    parts = []
    for h in range(3):
        num = a[:, 64 * h:64 * (h + 1)] if h < 2 else b[:, 0:64]
        parts.append(num / (b[:, 64 + h:65 + h] + 1e-16))
    return jnp.concatenate(parts, axis=1) + bias


def _t1_body(mp_ref, bias_ref, oh_ref, h1_ref, s1_ref, s2_ref, deg_ref):
    i = pl.program_id(0)
    h1 = jnp.maximum(_norm_msgs(mp_ref[...], bias_ref[...]), 0.0)
    h1_ref[...] = h1
    ohb = oh_ref[...]

    @pl.when(i == 0)
    def _():
        s1_ref[...] = jnp.zeros((8, 192), F32)
        s2_ref[...] = jnp.zeros((8, 192), F32)
        deg_ref[...] = jnp.zeros((8, 128), F32)

    dn = (((0,), (0,)), ((), ()))
    s1_ref[...] += lax.dot_general(ohb, h1, dn, preferred_element_type=F32)
    s2_ref[...] += lax.dot_general(ohb, h1 * h1, dn,
                                   preferred_element_type=F32)
    deg_ref[...] += lax.dot_general(ohb, jnp.ones((512, 128), F32), dn,
                                    preferred_element_type=F32)


def _t1(mp, bias1, oh):
    return pl.pallas_call(
        _t1_body, grid=(NB,),
        in_specs=[
            pl.BlockSpec((2, 2, 512, 128), lambda i: (0, 0, i, 0)),
            pl.BlockSpec((1, 192), lambda i: (0, 0)),
            pl.BlockSpec((512, 8), lambda i: (i, 0)),
        ],
        out_specs=[pl.BlockSpec((512, 192), lambda i: (i, 0)),
                   pl.BlockSpec((8, 192), lambda i: (0, 0)),
                   pl.BlockSpec((8, 192), lambda i: (0, 0)),
                   pl.BlockSpec((8, 128), lambda i: (0, 0))],
        out_shape=[jax.ShapeDtypeStruct((NP, 192), F32),
                   jax.ShapeDtypeStruct((8, 192), F32),
                   jax.ShapeDtypeStruct((8, 192), F32),
                   jax.ShapeDtypeStruct((8, 128), F32)],
    )(mp, bias1.reshape(1, 192), oh)


def _t2_body(h1_ref, s1_ref, s2_ref, deg_ref, oh_ref, nw_ref, nb_ref,
             wl_ref, bl_ref, wr_ref, br_ref, xl_ref, xr_ref):
    denom = jnp.maximum(deg_ref[:, 0:1] * 192.0, 1.0)
    mean = jnp.sum(s1_ref[...], axis=1, keepdims=True) / denom
    var = jnp.sum(s2_ref[...], axis=1, keepdims=True) / denom - mean * mean
    ohb = oh_ref[...]
    mean_n = jnp.dot(ohb, mean, preferred_element_type=F32)
    var_n = jnp.dot(ohb, var, preferred_element_type=F32)
    xn = (h1_ref[...] - mean_n) / jnp.sqrt(var_n + 1e-5)
    xn = xn * nw_ref[...] + nb_ref[...]
    xl_ref[...] = jnp.dot(xn, wl_ref[...],
                          preferred_element_type=F32) + bl_ref[...]
    xr_ref[...] = jnp.dot(xn, wr_ref[...],
                          preferred_element_type=F32) + br_ref[...]


def _t2(h1, s1, s2, deg, oh, nw, nb, wl, bl, wr, br):
    wo = wl.shape[1]
    return pl.pallas_call(
        _t2_body, grid=(NB,),
        in_specs=[
            pl.BlockSpec((512, 192), lambda i: (i, 0)),
            pl.BlockSpec((8, 192), lambda i: (0, 0)),
            pl.BlockSpec((8, 192), lambda i: (0, 0)),
            pl.BlockSpec((8, 128), lambda i: (0, 0)),
            pl.BlockSpec((512, 8), lambda i: (i, 0)),
            pl.BlockSpec((1, 192), lambda i: (0, 0)),
            pl.BlockSpec((1, 192), lambda i: (0, 0)),
            pl.BlockSpec((192, wo), lambda i: (0, 0)),
            pl.BlockSpec((1, wo), lambda i: (0, 0)),
            pl.BlockSpec((192, wo), lambda i: (0, 0)),
            pl.BlockSpec((1, wo), lambda i: (0, 0)),
        ],
        out_specs=[pl.BlockSpec((512, wo), lambda i: (i, 0)),
                   pl.BlockSpec((512, wo), lambda i: (i, 0))],
        out_shape=[jax.ShapeDtypeStruct((NP, wo), F32),
                   jax.ShapeDtypeStruct((NP, wo), F32)],
    )(h1, s1, s2, deg, oh, nw.reshape(1, 192), nb.reshape(1, 192),
      wl, bl.reshape(1, wo), wr, br.reshape(1, wo))


def _t3_body(mp_ref, bias_ref, oh_ref, wl_ref, bl_ref, wr_ref, br_ref,
             xl_ref, xr_ref, pool_ref):
    i = pl.program_id(0)
    h2 = _norm_msgs(mp_ref[...], bias_ref[...])
    ohb = oh_ref[...]

    @pl.when(i == 0)
    def _():
        pool_ref[...] = jnp.zeros((8, 192), F32)

    dn = (((0,), (0,)), ((), ()))
    pool_ref[...] += lax.dot_general(ohb, h2, dn, preferred_element_type=F32)
    cc = jnp.maximum(h2, 0.0)
    xl_ref[...] = jnp.dot(cc, wl_ref[...],
                          preferred_element_type=F32) + bl_ref[...]
    xr_ref[...] = jnp.dot(cc, wr_ref[...],
                          preferred_element_type=F32) + br_ref[...]


def _t3(mp, bias2, oh, wl, bl, wr, br):
    return pl.pallas_call(
        _t3_body, grid=(NB,),
        in_specs=[
            pl.BlockSpec((2, 2, 512, 128), lambda i: (0, 0, i, 0)),
            pl.BlockSpec((1, 192), lambda i: (0, 0)),
            pl.BlockSpec((512, 8), lambda i: (i, 0)),
            pl.BlockSpec((192, 384), lambda i: (0, 0)),
            pl.BlockSpec((1, 384), lambda i: (0, 0)),
            pl.BlockSpec((192, 384), lambda i: (0, 0)),
            pl.BlockSpec((1, 384), lambda i: (0, 0)),
        ],
        out_specs=[pl.BlockSpec((512, 384), lambda i: (i, 0)),
                   pl.BlockSpec((512, 384), lambda i: (i, 0)),
                   pl.BlockSpec((8, 192), lambda i: (0, 0))],
        out_shape=[jax.ShapeDtypeStruct((NP, 384), F32),
                   jax.ShapeDtypeStruct((NP, 384), F32),
                   jax.ShapeDtypeStruct((8, 192), F32)],
    )(mp, bias2.reshape(1, 192), oh,
      wl, bl.reshape(1, 384), wr, br.reshape(1, 384))


def _t4_body(mp_ref, bias_ref, pool_ref, deg_ref, cw_ref, cb_ref,
             color_ref, cls_ref):
    i = pl.program_id(0)
    m = mp_ref[...]
    sx = m[0, 3] + m[1, 3]
    acc = jnp.zeros((512, 128), F32)
    for h in range(3):
        acc += (m[0, h] + m[1, h]) / (sx[:, h:h + 1] + 1e-16)
    color_ref[...] = acc * (1.0 / 3.0) + bias_ref[...]

    @pl.when(i == 0)
    def _():
        cnt = jnp.maximum(deg_ref[:, 0:1], 1.0)
        pooled = pool_ref[...] / cnt
        cls_ref[...] = jnp.dot(pooled, cw_ref[...],
                               preferred_element_type=F32) + cb_ref[...]


def _t4(mp, bias3, pool, deg, cw_pad, cb_pad):
    return pl.pallas_call(
        _t4_body, grid=(NB,),
        in_specs=[
            pl.BlockSpec((2, 4, 512, 128), lambda i: (0, 0, i, 0)),
            pl.BlockSpec((1, 128), lambda i: (0, 0)),
            pl.BlockSpec((8, 192), lambda i: (0, 0)),
            pl.BlockSpec((8, 128), lambda i: (0, 0)),
            pl.BlockSpec((192, 128), lambda i: (0, 0)),
            pl.BlockSpec((1, 128), lambda i: (0, 0)),
        ],
        out_specs=[pl.BlockSpec((512, 128), lambda i: (i, 0)),
                   pl.BlockSpec((8, 128), lambda i: (0, 0))],
        out_shape=[jax.ShapeDtypeStruct((NP, 128), F32),
                   jax.ShapeDtypeStruct((8, 128), F32)],
    )(mp, bias3.reshape(1, 128), pool, deg, cw_pad, cb_pad.reshape(1, 128))


def _pad_nodes(a, rows=NP):
    pad = [(0, 0)] * a.ndim
    pad[-2] = (0, rows - a.shape[-2])
    return jnp.pad(a, pad)


def _pad_cols(a, w):
    if a.ndim == 1:
        return jnp.concatenate([a, jnp.zeros((w - a.shape[0],), a.dtype)])
    return jnp.concatenate(
        [a, jnp.zeros((a.shape[0], w - a.shape[1]), a.dtype)], 1)


def _edge_phase(xl, xr, src2d, dst2d, att_flat, oc, z128):
    xj, xi = _make_gather(xl.shape[1])(xl, xr, src2d, dst2d)
    slabs = _edge(xj, xi, att_flat, oc)
    mp = _make_scatter(len(slabs))(*slabs, dst2d, z128)
    return _pad_nodes(mp)


def kernel(x, edge_index, batch, w_l1, b_l1, w_r1, b_r1, att1, bias1,
           w_l2, b_l2, w_r2, b_r2, att2, bias2,
           w_l3, b_l3, w_r3, b_r3, att3, bias3,
           norm_w, norm_b, cls_w, cls_b):
    idt = edge_index.dtype
    loop = jnp.arange(N, dtype=idt)
    src = jnp.concatenate([edge_index[0], loop])
    dst = jnp.concatenate([edge_index[1], loop])
    padi = jnp.full((EP - E,), N, dtype=idt)
    src2d = jnp.concatenate([src, padi]).astype(jnp.int32).reshape(EP // 128,
                                                                   128)
    dst2d = jnp.concatenate([dst, padi]).astype(jnp.int32).reshape(EP // 128,
                                                                   128)
    xp = _pad_nodes(x)
    batch_pad = jnp.concatenate(
        [batch, jnp.full((NP - N,), 8, dtype=batch.dtype)])
    oh = (batch_pad[:, None] == jnp.arange(8, dtype=batch.dtype)[None, :]
          ).astype(F32)
    z128 = jnp.zeros((NA, 128), F32)

    xl1, xr1 = _lin2(xp, _pad_cols(w_l1, 256), _pad_cols(b_l1, 256),
                     _pad_cols(w_r1, 256), _pad_cols(b_r1, 256))
    mp1 = _edge_phase(xl1, xr1, src2d, dst2d,
                      _pad_cols(att1.reshape(1, 192), 256), 64, z128)
    h1, s1, s2, deg = _t1(mp1, bias1, oh)

    xl2, xr2 = _t2(h1, s1, s2, deg, oh, norm_w, norm_b,
                   _pad_cols(w_l2, 256), _pad_cols(b_l2, 256),
                   _pad_cols(w_r2, 256), _pad_cols(b_r2, 256))
    mp2 = _edge_phase(xl2, xr2, src2d, dst2d,
                      _pad_cols(att2.reshape(1, 192), 256), 64, z128)

    xl3, xr3, pool = _t3(mp2, bias2, oh, w_l3, b_l3, w_r3, b_r3)
    mp3 = _edge_phase(xl3, xr3, src2d, dst2d, att3.reshape(1, 384), 128,
                      z128)

    cw_pad = jnp.concatenate([cls_w, jnp.zeros((192, 118), F32)], axis=1)
    cb_pad = jnp.concatenate([cls_b, jnp.zeros((118,), F32)])
    color_full, cls_full = _t4(mp3, bias3, pool, deg, cw_pad, cb_pad)
    return cls_full[:8, :10], color_full[:N]
